# Initial kernel scaffold; baseline (speedup 1.0000x reference)
import jax
import jax.numpy as jnp
from jax.experimental import pallas as pl


def kernel(x_e, edge_index, rel, edge_index_all, rel_all, hw1_W, hw2_W, ge_ah1, ge_ah2, ge_at1, ge_at2, ge_wh, ge_wt, gr_ah, gr_at, gr_ar, g_ai, g_aj):
    raise NotImplementedError("write your pallas kernel here")


# speedup vs baseline: 1.7325x; 1.7325x over previous
import functools
import jax
import jax.numpy as jnp
from jax.experimental import pallas as pl
from jax.experimental.pallas import tpu as pltpu

N = 10000
E = 320000
EH = 128
RH = 32
NREL = 100

lrelu = jax.nn.leaky_relu


def _finish_body(x_ref, gro_ref, accA_ref, q_ref, s_ref, gro100_ref, o_ref):
    s = s_ref[0] + s_ref[1]
    denom = s + 1e-16
    nA = accA_ref[0] + accA_ref[1]
    q = q_ref[0] + q_ref[1]
    nB = jnp.dot(q, gro100_ref[...], preferred_element_type=jnp.float32)
    o_ref[:, :EH] = x_ref[...]
    o_ref[:, EH:2 * EH] = gro_ref[...]
    o_ref[:, 2 * EH:3 * EH] = lrelu(nA / denom)
    o_ref[:, 3 * EH:] = lrelu(nB / denom)


def _finish(x, gro, accA, q, s, gro100):
    BR = 2000
    grid = (N // BR,)
    return pl.pallas_call(
        _finish_body,
        grid=grid,
        in_specs=[
            pl.BlockSpec((BR, EH), lambda r: (r, 0)),
            pl.BlockSpec((BR, EH), lambda r: (r, 0)),
            pl.BlockSpec((2, BR, EH), lambda r: (0, r, 0)),
            pl.BlockSpec((2, BR, NREL), lambda r: (0, r, 0)),
            pl.BlockSpec((2, BR, 1), lambda r: (0, r, 0)),
            pl.BlockSpec((NREL, EH), lambda r: (0, 0)),
        ],
        out_specs=pl.BlockSpec((BR, 4 * EH), lambda r: (r, 0)),
        out_shape=jax.ShapeDtypeStruct((N, 4 * EH), jnp.float32),
    )(x, gro, accA, q, s, gro100)


def kernel(x_e, edge_index, rel, edge_index_all, rel_all, hw1_W, hw2_W,
           ge_ah1, ge_ah2, ge_at1, ge_at2, ge_wh, ge_wt, gr_ah, gr_at,
           gr_ar, g_ai, g_aj):
    j_all, i_all = edge_index_all[0], edge_index_all[1]
    h, t = edge_index[0], edge_index[1]

    deg = jax.ops.segment_sum(jnp.ones(E, jnp.float32), i_all, num_segments=N)
    dis = jax.lax.rsqrt(deg)

    def gcn(x):
        y = dis[:, None] * x
        acc = jax.ops.segment_sum(y[j_all], i_all, num_segments=N)
        return jax.nn.relu(dis[:, None] * acc)

    def highway(x1, x2, W):
        gate = lrelu(x1 @ W + 1e-08)
        return lrelu(gate * x2 + (1.0 - gate) * x1)

    x = highway(x_e, gcn(x_e), hw1_W)
    x = highway(x, gcn(x), hw2_W)

    x_r_h = x @ ge_wh
    x_r_t = x @ ge_wt
    u1 = x_r_h @ ge_ah1; u2 = x_r_t @ ge_ah2
    v1 = x_r_h @ ge_at1; v2 = x_r_t @ ge_at2
    M1 = lrelu(jnp.max(u1) + jnp.max(u2))
    M2 = lrelu(jnp.max(v1) + jnp.max(v2))
    w1 = jnp.exp(lrelu(u1[h] + u2[t]) - M1)
    w2 = jnp.exp(lrelu(v1[h] + v2[t]) - M2)
    s1 = jax.ops.segment_sum(w1, rel, num_segments=NREL)
    s2 = jax.ops.segment_sum(w2, rel, num_segments=NREL)
    n1 = jax.ops.segment_sum(w1[:, None] * x_r_h[h], rel, num_segments=NREL)
    n2 = jax.ops.segment_sum(w2[:, None] * x_r_t[t], rel, num_segments=NREL)
    x_r = n1 / (s1[:, None] + 1e-16) + n2 / (s2[:, None] + 1e-16)

    eh_tab = x @ gr_ah
    er_tab = x_r @ gr_ar
    M3 = lrelu(jnp.max(eh_tab) + jnp.max(er_tab))
    w3 = jnp.exp(lrelu(eh_tab[h] + er_tab[rel]) - M3)
    s3 = jax.ops.segment_sum(w3, rel, num_segments=NREL)
    n3 = jax.ops.segment_sum(w3[:, None] * x[t], rel, num_segments=NREL)
    gro100 = n3 / (s3[:, None] + 1e-16)
    gro = jnp.zeros((N, EH), jnp.float32).at[:NREL].set(gro100)

    ei_tab = x @ g_ai[:EH] + gro @ g_ai[EH:]
    ej_tab = x @ g_aj[:EH] + gro @ g_aj[EH:]
    M4 = lrelu(jnp.max(ei_tab) + jnp.max(ej_tab))
    w4 = jnp.exp(lrelu(ei_tab[i_all] + ej_tab[j_all]) - M4)
    s4 = jax.ops.segment_sum(w4, i_all, num_segments=N)
    nA = jax.ops.segment_sum(w4[:, None] * x[j_all], i_all, num_segments=N)
    mask = j_all < NREL
    flat = jnp.where(mask, i_all * NREL + j_all, N * NREL)
    Qf = jax.ops.segment_sum(w4, flat, num_segments=N * NREL + 1)
    Q = Qf[:N * NREL].reshape(N, NREL)

    accA = jnp.stack([nA, jnp.zeros_like(nA)])
    q = jnp.stack([Q, jnp.zeros_like(Q)])
    s = jnp.stack([s4, jnp.zeros_like(s4)])[:, :, None]
    return _finish(x, gro, accA, q, s, gro100)


# speedup vs baseline: 1.9198x; 1.1081x over previous
import functools
import jax
import jax.numpy as jnp
from jax import lax
from jax.experimental import pallas as pl
from jax.experimental.pallas import tpu as pltpu
from jax.experimental.pallas import tpu_sc as plsc

N = 10000
E = 320000
EH = 128
RH = 32
NREL = 100

NCORE = 2
NSUB = 16
NW = NCORE * NSUB
EPW = E // NW
K = 80
NCHUNK = EPW // K
SPAD = 10240

lrelu = jax.nn.leaky_relu

@functools.cache
def _mesh():
    return plsc.VectorSubcoreMesh(core_axis_name="c", subcore_axis_name="s")


def _zero_vmem(ref, shape):
    z = jnp.zeros((16,), jnp.float32)
    if len(shape) == 1:
        for k0 in range(shape[0] // 16):
            ref[pl.ds(k0 * 16, 16)] = z
    else:
        for r in range(shape[0]):
            for c0 in range(shape[1] // 16):
                ref[r, pl.ds(c0 * 16, 16)] = z


@functools.cache
def _deg_sc():
    return functools.partial(
        pl.kernel,
        out_type=jax.ShapeDtypeStruct((2 * SPAD,), jnp.float32),
        mesh=_mesh(),
        scratch_types=[
            pltpu.VMEM((K,), jnp.int32),
            pltpu.VMEM((K,), jnp.float32),
            pltpu.VMEM((640,), jnp.float32),
            pltpu.VMEM_SHARED((SPAD,), jnp.float32),
            pltpu.SemaphoreType.DMA,
        ],
    )(_deg_body)


def _deg_body(i_hbm, out_hbm, idx_v, ones_v, zb_v, deg_sh, sem):
    c = lax.axis_index("c")
    s = lax.axis_index("s")
    wid = c * NSUB + s
    one = jnp.ones((16,), jnp.float32)
    for g in range(K // 16):
        ones_v[pl.ds(g * 16, 16)] = one
    _zero_vmem(zb_v, (640,))
    pltpu.sync_copy(zb_v, deg_sh.at[pl.ds(s * 640, 640)])
    plsc.subcore_barrier()

    def body(cc, carry):
        base = wid * EPW + cc * K
        pltpu.sync_copy(i_hbm.at[pl.ds(base, K)], idx_v)
        pltpu.sync_copy(ones_v, deg_sh.at[idx_v], add=True)
        return carry

    lax.fori_loop(0, NCHUNK, body, 0)
    plsc.subcore_barrier()
    pltpu.sync_copy(deg_sh.at[pl.ds(s * 640, 640)], zb_v)
    pltpu.sync_copy(zb_v, out_hbm.at[pl.ds(c * SPAD + s * 640, 640)])


@functools.cache
def _gcn_sc():
    return functools.partial(
        pl.kernel,
        out_type=jax.ShapeDtypeStruct((2, N, EH), jnp.float32),
        mesh=_mesh(),
        scratch_types=[
            pltpu.VMEM((K,), jnp.int32),
            pltpu.VMEM((K,), jnp.int32),
            pltpu.VMEM((K, EH), jnp.float32),
            pltpu.VMEM((25, EH), jnp.float32),
            pltpu.VMEM_SHARED((N, EH), jnp.float32),
            pltpu.SemaphoreType.DMA,
        ],
    )(_gcn_body)


def _gcn_body(y_hbm, j_hbm, i_hbm, out_hbm, idxj_v, idxi_v, rows_v, zb_v,
              acc_sh, sem):
    c = lax.axis_index("c")
    s = lax.axis_index("s")
    wid = c * NSUB + s
    _zero_vmem(zb_v, (25, EH))

    def zbody(q, carry):
        pltpu.sync_copy(zb_v, acc_sh.at[pl.ds(s * 625 + q * 25, 25)])
        return carry

    lax.fori_loop(0, 25, zbody, 0)
    plsc.subcore_barrier()

    def body(cc, carry):
        base = wid * EPW + cc * K
        pltpu.sync_copy(j_hbm.at[pl.ds(base, K)], idxj_v)
        pltpu.sync_copy(i_hbm.at[pl.ds(base, K)], idxi_v)
        pltpu.async_copy(y_hbm.at[idxj_v], rows_v, sem).wait()
        pltpu.sync_copy(rows_v, acc_sh.at[idxi_v], add=True)
        return carry

    lax.fori_loop(0, NCHUNK, body, 0)
    plsc.subcore_barrier()

    @pl.when(s == 0)
    def _():
        pltpu.sync_copy(acc_sh, out_hbm.at[c])


def _finish_body(x_ref, gro_ref, accA_ref, q_ref, s_ref, gro100_ref, o_ref):
    s = s_ref[0] + s_ref[1]
    denom = s + 1e-16
    nA = accA_ref[0] + accA_ref[1]
    q = q_ref[0] + q_ref[1]
    nB = jnp.dot(q, gro100_ref[...], preferred_element_type=jnp.float32)
    o_ref[:, :EH] = x_ref[...]
    o_ref[:, EH:2 * EH] = gro_ref[...]
    o_ref[:, 2 * EH:3 * EH] = lrelu(nA / denom)
    o_ref[:, 3 * EH:] = lrelu(nB / denom)


def _finish(x, gro, accA, q, s, gro100):
    BR = 2000
    grid = (N // BR,)
    return pl.pallas_call(
        _finish_body,
        grid=grid,
        in_specs=[
            pl.BlockSpec((BR, EH), lambda r: (r, 0)),
            pl.BlockSpec((BR, EH), lambda r: (r, 0)),
            pl.BlockSpec((2, BR, EH), lambda r: (0, r, 0)),
            pl.BlockSpec((2, BR, NREL), lambda r: (0, r, 0)),
            pl.BlockSpec((2, BR, 1), lambda r: (0, r, 0)),
            pl.BlockSpec((NREL, EH), lambda r: (0, 0)),
        ],
        out_specs=pl.BlockSpec((BR, 4 * EH), lambda r: (r, 0)),
        out_shape=jax.ShapeDtypeStruct((N, 4 * EH), jnp.float32),
    )(x, gro, accA, q, s, gro100)


def kernel(x_e, edge_index, rel, edge_index_all, rel_all, hw1_W, hw2_W,
           ge_ah1, ge_ah2, ge_at1, ge_at2, ge_wh, ge_wt, gr_ah, gr_at,
           gr_ar, g_ai, g_aj):
    j_all, i_all = edge_index_all[0], edge_index_all[1]
    h, t = edge_index[0], edge_index[1]

    deg_p = _deg_sc()(i_all).reshape(2, SPAD)
    deg = deg_p[0, :N] + deg_p[1, :N]
    dis = jax.lax.rsqrt(deg)

    def gcn(x):
        y = dis[:, None] * x
        accp = _gcn_sc()(y, j_all, i_all)
        return jax.nn.relu(dis[:, None] * (accp[0] + accp[1]))

    def highway(x1, x2, W):
        gate = lrelu(x1 @ W + 1e-08)
        return lrelu(gate * x2 + (1.0 - gate) * x1)

    x = highway(x_e, gcn(x_e), hw1_W)
    x = highway(x, gcn(x), hw2_W)

    x_r_h = x @ ge_wh
    x_r_t = x @ ge_wt
    u1 = x_r_h @ ge_ah1; u2 = x_r_t @ ge_ah2
    v1 = x_r_h @ ge_at1; v2 = x_r_t @ ge_at2
    M1 = lrelu(jnp.max(u1) + jnp.max(u2))
    M2 = lrelu(jnp.max(v1) + jnp.max(v2))
    w1 = jnp.exp(lrelu(u1[h] + u2[t]) - M1)
    w2 = jnp.exp(lrelu(v1[h] + v2[t]) - M2)
    s1 = jax.ops.segment_sum(w1, rel, num_segments=NREL)
    s2 = jax.ops.segment_sum(w2, rel, num_segments=NREL)
    n1 = jax.ops.segment_sum(w1[:, None] * x_r_h[h], rel, num_segments=NREL)
    n2 = jax.ops.segment_sum(w2[:, None] * x_r_t[t], rel, num_segments=NREL)
    x_r = n1 / (s1[:, None] + 1e-16) + n2 / (s2[:, None] + 1e-16)

    eh_tab = x @ gr_ah
    er_tab = x_r @ gr_ar
    M3 = lrelu(jnp.max(eh_tab) + jnp.max(er_tab))
    w3 = jnp.exp(lrelu(eh_tab[h] + er_tab[rel]) - M3)
    s3 = jax.ops.segment_sum(w3, rel, num_segments=NREL)
    n3 = jax.ops.segment_sum(w3[:, None] * x[t], rel, num_segments=NREL)
    gro100 = n3 / (s3[:, None] + 1e-16)
    gro = jnp.zeros((N, EH), jnp.float32).at[:NREL].set(gro100)

    ei_tab = x @ g_ai[:EH] + gro @ g_ai[EH:]
    ej_tab = x @ g_aj[:EH] + gro @ g_aj[EH:]
    M4 = lrelu(jnp.max(ei_tab) + jnp.max(ej_tab))
    w4 = jnp.exp(lrelu(ei_tab[i_all] + ej_tab[j_all]) - M4)
    s4 = jax.ops.segment_sum(w4, i_all, num_segments=N)
    nA = jax.ops.segment_sum(w4[:, None] * x[j_all], i_all, num_segments=N)
    mask = j_all < NREL
    flat = jnp.where(mask, i_all * NREL + j_all, N * NREL)
    Qf = jax.ops.segment_sum(w4, flat, num_segments=N * NREL + 1)
    Q = Qf[:N * NREL].reshape(N, NREL)

    accA = jnp.stack([nA, jnp.zeros_like(nA)])
    q = jnp.stack([Q, jnp.zeros_like(Q)])
    s = jnp.stack([s4, jnp.zeros_like(s4)])[:, :, None]
    return _finish(x, gro, accA, q, s, gro100)
